# baseline (speedup 1.0000x reference)
import jax
import jax.numpy as jnp
from jax import lax
from jax.experimental import pallas as pl
from jax.experimental.pallas import tpu as pltpu

_BM = 13952


def _mlp_kernel(x_ref, w1_ref, b1_ref, w2t_ref, b2_ref, ot_ref):
    x = x_ref[...].astype(jnp.bfloat16)
    w1 = w1_ref[...].astype(jnp.bfloat16)
    h = jnp.dot(x, w1, preferred_element_type=jnp.float32).astype(jnp.bfloat16)
    h = jnp.maximum(h + b1_ref[...].astype(jnp.bfloat16),
                    jnp.asarray(0, jnp.bfloat16))
    w2t = w2t_ref[...].astype(jnp.bfloat16)
    ot = lax.dot_general(w2t, h, (((1,), (1,)), ((), ())),
                         preferred_element_type=jnp.float32)
    ot_ref[...] = ot + b2_ref[...]


def kernel(features, W1, b1, W2, b2):
    n, d = features.shape
    h = W1.shape[1]
    c = W2.shape[1]
    out_t = pl.pallas_call(
        _mlp_kernel,
        grid=(pl.cdiv(n, _BM),),
        in_specs=[
            pl.BlockSpec((_BM, d), lambda i: ((i + 3) % 4, 0)),
            pl.BlockSpec((d, h), lambda i: (0, 0)),
            pl.BlockSpec((1, h), lambda i: (0, 0)),
            pl.BlockSpec((c, d), lambda i: (0, 0)),
            pl.BlockSpec((c, 1), lambda i: (0, 0)),
        ],
        out_specs=pl.BlockSpec((c, _BM), lambda i: (0, (i + 3) % 4)),
        out_shape=jax.ShapeDtypeStruct((c, n), jnp.float32),
        compiler_params=pltpu.CompilerParams(
            dimension_semantics=("parallel",),
            vmem_limit_bytes=100 * 1024 * 1024,
        ),
    )(features, W1, b1.reshape(1, h), W2.T, b2.reshape(c, 1))
    return out_t.T


# speedup vs baseline: 1.0118x; 1.0118x over previous
import jax
import jax.numpy as jnp
from jax import lax
from jax.experimental import pallas as pl
from jax.experimental.pallas import tpu as pltpu

_BM = 12544


def _mlp_kernel(x_ref, w1_ref, b1_ref, w2t_ref, b2_ref, ot_ref):
    x = x_ref[...].astype(jnp.bfloat16)
    w1 = w1_ref[...].astype(jnp.bfloat16)
    h = jnp.dot(x, w1, preferred_element_type=jnp.float32).astype(jnp.bfloat16)
    h = jnp.maximum(h + b1_ref[...].astype(jnp.bfloat16),
                    jnp.asarray(0, jnp.bfloat16))
    w2t = w2t_ref[...].astype(jnp.bfloat16)
    ot = lax.dot_general(w2t, h, (((1,), (1,)), ((), ())),
                         preferred_element_type=jnp.float32)
    ot_ref[...] = ot + b2_ref[...]


def kernel(features, W1, b1, W2, b2):
    n, d = features.shape
    h = W1.shape[1]
    c = W2.shape[1]
    out_t = pl.pallas_call(
        _mlp_kernel,
        grid=(pl.cdiv(n, _BM),),
        in_specs=[
            pl.BlockSpec((_BM, d), lambda i: (i, 0)),
            pl.BlockSpec((d, h), lambda i: (0, 0)),
            pl.BlockSpec((1, h), lambda i: (0, 0)),
            pl.BlockSpec((c, d), lambda i: (0, 0)),
            pl.BlockSpec((c, 1), lambda i: (0, 0)),
        ],
        out_specs=pl.BlockSpec((c, _BM), lambda i: (0, i)),
        out_shape=jax.ShapeDtypeStruct((c, n), jnp.float32),
        compiler_params=pltpu.CompilerParams(
            dimension_semantics=("parallel",),
            vmem_limit_bytes=100 * 1024 * 1024,
        ),
    )(features, W1, b1.reshape(1, h), W2.T, b2.reshape(c, 1))
    return out_t.T
